# baseline (speedup 1.0000x reference)
import functools

import jax
import jax.numpy as jnp
from jax import lax
from jax.experimental import pallas as pl
from jax.experimental.pallas import tpu as pltpu
from jax.experimental.pallas import tpu_sc as plsc

B = 8
H = 384
W = 384
N = 1024
NC = 2
NS = 16
ROWS_PER_TILE = H // NS
SEG_PER_TILE = ROWS_PER_TILE * W
PAIR_VECS = SEG_PER_TILE // 16
UNROLL = 8
J_ITERS = PAIR_VECS // UNROLL
NUM_IDX = 2 * PAIR_VECS * 16
ACC_PAD = 64
ACC = N * N + ACC_PAD
IMGS_PER_CORE = B // NC
SLICE = N * N // NS
ZCHUNK = 16384


def _adj_body(seg_hbm, out_hbm, segbuf, idx_buf, ones_buf, zbuf, acc):
    core = lax.axis_index("c")
    sid = lax.axis_index("s")
    iota = lax.iota(jnp.int32, 16)
    ones_v = jnp.full((16,), 1.0, dtype=jnp.float32)
    zero_v = jnp.zeros((16,), dtype=jnp.float32)

    def fill_ones(j, carry):
        ones_buf[pl.ds(j * 16, 16)] = ones_v
        return carry

    lax.fori_loop(0, NUM_IDX // 16, fill_ones, 0)

    def fill_zero(i, carry):
        zbuf[pl.ds(i * 16, 16)] = zero_v
        return carry

    lax.fori_loop(0, ZCHUNK // 16, fill_zero, 0)

    my_slice = pl.multiple_of(sid * SLICE, 8)

    for i in range(IMGS_PER_CORE):
        img = core * IMGS_PER_CORE + i

        seg_start = pl.multiple_of(sid * SEG_PER_TILE, 8)
        pltpu.sync_copy(
            seg_hbm.at[img, pl.ds(seg_start, SEG_PER_TILE)],
            segbuf.at[pl.ds(0, SEG_PER_TILE)],
        )

        for k in range(SLICE // ZCHUNK):
            pltpu.sync_copy(zbuf, acc.at[pl.ds(my_slice + k * ZCHUNK, ZCHUNK)])

        @pl.when(sid == 0)
        def _zero_pad():
            pltpu.sync_copy(zbuf.at[pl.ds(0, ACC_PAD)], acc.at[pl.ds(N * N, ACC_PAD)])

        def pair_chunks(j, carry):
            for u in range(UNROLL):
                p = j * (UNROLL * 16) + u * 16
                left = segbuf[pl.ds(pl.multiple_of(p, 16), 16)]
                right = segbuf[pl.ds(p + 1, 16)]
                pos = p + iota
                valid = (left != right) & (lax.rem(pos, W) != (W - 1))
                trash = N * N + iota
                fwd = jnp.where(valid, left * N + right, trash)
                bwd = jnp.where(valid, right * N + left, trash)
                idx_buf[pl.ds(p, 16)] = fwd
                idx_buf[pl.ds(SEG_PER_TILE + p, 16)] = bwd
            return carry

        lax.fori_loop(0, J_ITERS, pair_chunks, 0)

        plsc.subcore_barrier()

        pltpu.sync_copy(ones_buf, acc.at[idx_buf], add=True)

        plsc.subcore_barrier()

        def copy_row(r, carry):
            row = sid * (N // NS) + r
            pltpu.sync_copy(
                acc.at[pl.ds(pl.multiple_of(row * N, 8), N)],
                out_hbm.at[img, row, :],
            )
            return carry

        lax.fori_loop(0, N // NS, copy_row, 0)


@jax.jit
def _adjacency(seg_flat):
    mesh = plsc.VectorSubcoreMesh(
        core_axis_name="c", subcore_axis_name="s", num_cores=NC, num_subcores=NS
    )
    run = functools.partial(
        pl.kernel,
        out_type=jax.ShapeDtypeStruct((B, N, N), jnp.float32),
        mesh=mesh,
        scratch_types=[
            pltpu.VMEM((SEG_PER_TILE + 8,), jnp.int32),
            pltpu.VMEM((NUM_IDX,), jnp.int32),
            pltpu.VMEM((NUM_IDX,), jnp.float32),
            pltpu.VMEM((ZCHUNK,), jnp.float32),
            pltpu.VMEM_SHARED((ACC,), jnp.float32),
        ],
    )(_adj_body)
    return run(seg_flat)


def kernel(segments):
    if segments.ndim == 4:
        segments = segments[:, 0]
    seg_flat = segments.reshape(B, H * W).astype(jnp.int32)
    return _adjacency(seg_flat)


# speedup vs baseline: 2.1072x; 2.1072x over previous
import functools

import jax
import jax.numpy as jnp
from jax import lax
from jax.experimental import pallas as pl
from jax.experimental.pallas import tpu as pltpu
from jax.experimental.pallas import tpu_sc as plsc

B = 8
H = 384
W = 384
N = 1024
NC = 2
NS = 16
ROWS_PER_TILE = H // NS
SEG_PER_TILE = ROWS_PER_TILE * W
PAIR_VECS = SEG_PER_TILE // 16
UNROLL = 8
J_ITERS = PAIR_VECS // UNROLL
NUM_IDX = SEG_PER_TILE
ACC_PAD = 64
ACC = N * N + ACC_PAD
IMGS_PER_CORE = B // NC
SLICE = N * N // NS
ZCHUNK = 8192


def _adj_body(seg_hbm, out_hbm, segbuf, fwd_a, bwd_a, fwd_b, bwd_b, ones_buf, zbuf, acc, sem):
    core = lax.axis_index("c")
    sid = lax.axis_index("s")
    iota = lax.iota(jnp.int32, 16)
    ones_v = jnp.full((16,), 1.0, dtype=jnp.float32)
    zero_v = jnp.zeros((16,), dtype=jnp.float32)

    def fill_ones(j, carry):
        ones_buf[pl.ds(j * 16, 16)] = ones_v
        return carry

    lax.fori_loop(0, NUM_IDX // 16, fill_ones, 0)

    def fill_zero(i, carry):
        zbuf[pl.ds(i * 16, 16)] = zero_v
        return carry

    lax.fori_loop(0, ZCHUNK // 16, fill_zero, 0)

    my_slice = pl.multiple_of(sid * SLICE, 8)
    seg_start = pl.multiple_of(sid * SEG_PER_TILE, 8)

    def zero_my_slice():
        for k in range(SLICE // ZCHUNK):
            pltpu.sync_copy(zbuf, acc.at[pl.ds(my_slice + k * ZCHUNK, ZCHUNK)])

        @pl.when(sid == 0)
        def _zero_pad():
            pltpu.sync_copy(
                zbuf.at[pl.ds(0, ACC_PAD)], acc.at[pl.ds(N * N, ACC_PAD)]
            )

    def stage_and_compute(img, fwd_buf, bwd_buf):
        pltpu.sync_copy(
            seg_hbm.at[img, pl.ds(seg_start, SEG_PER_TILE)],
            segbuf.at[pl.ds(0, SEG_PER_TILE)],
        )

        def pair_chunks(j, carry):
            for u in range(UNROLL):
                p = j * (UNROLL * 16) + u * 16
                left = segbuf[pl.ds(pl.multiple_of(p, 16), 16)]
                right = segbuf[pl.ds(p + 1, 16)]
                pos = p + iota
                valid = (left != right) & (lax.rem(pos, W) != (W - 1))
                trash = N * N + iota
                fwd = jnp.where(valid, left * N + right, trash)
                bwd = jnp.where(valid, right * N + left, trash)
                fwd_buf[pl.ds(p, 16)] = fwd
                bwd_buf[pl.ds(p, 16)] = bwd
            return carry

        lax.fori_loop(0, J_ITERS, pair_chunks, 0)

    idx_bufs = [(fwd_a, bwd_a), (fwd_b, bwd_b)]
    stage_and_compute(core * IMGS_PER_CORE, fwd_a, bwd_a)
    zero_my_slice()
    plsc.subcore_barrier()

    for i in range(IMGS_PER_CORE):
        img = core * IMGS_PER_CORE + i

        cur_f, cur_b = idx_bufs[i % 2]
        scat_f = pltpu.async_copy(ones_buf, acc.at[cur_f], sem, add=True)
        scat_b = pltpu.async_copy(ones_buf, acc.at[cur_b], sem, add=True)
        if i + 1 < IMGS_PER_CORE:
            nxt_f, nxt_b = idx_bufs[(i + 1) % 2]
            stage_and_compute(img + 1, nxt_f, nxt_b)
        scat_f.wait()
        scat_b.wait()

        plsc.subcore_barrier()

        pltpu.sync_copy(
            acc.at[pl.ds(my_slice, SLICE)],
            out_hbm.at[img, pl.ds(my_slice, SLICE)],
        )

        if i + 1 < IMGS_PER_CORE:
            zero_my_slice()
            plsc.subcore_barrier()


@jax.jit
def _adjacency(seg_flat):
    mesh = plsc.VectorSubcoreMesh(
        core_axis_name="c", subcore_axis_name="s", num_cores=NC, num_subcores=NS
    )
    run = functools.partial(
        pl.kernel,
        out_type=jax.ShapeDtypeStruct((B, N * N), jnp.float32),
        mesh=mesh,
        scratch_types=[
            pltpu.VMEM((SEG_PER_TILE + 8,), jnp.int32),
            pltpu.VMEM((NUM_IDX,), jnp.int32),
            pltpu.VMEM((NUM_IDX,), jnp.int32),
            pltpu.VMEM((NUM_IDX,), jnp.int32),
            pltpu.VMEM((NUM_IDX,), jnp.int32),
            pltpu.VMEM((NUM_IDX,), jnp.float32),
            pltpu.VMEM((ZCHUNK,), jnp.float32),
            pltpu.VMEM_SHARED((ACC,), jnp.float32),
            pltpu.SemaphoreType.DMA,
        ],
    )(_adj_body)
    return run(seg_flat)


def kernel(segments):
    if segments.ndim == 4:
        segments = segments[:, 0]
    seg_flat = segments.reshape(B, H * W).astype(jnp.int32)
    out = _adjacency(seg_flat)
    return out.reshape(B, N, N)


# speedup vs baseline: 2.7869x; 1.3226x over previous
import functools

import jax
import jax.numpy as jnp
from jax import lax
from jax.experimental import pallas as pl
from jax.experimental.pallas import tpu as pltpu
from jax.experimental.pallas import tpu_sc as plsc

B = 8
H = 384
W = 384
N = 1024
NC = 2
NS = 16
ROWS_PER_TILE = H // NS
SEG_PER_TILE = ROWS_PER_TILE * W
PAIR_VECS = SEG_PER_TILE // 16
UNROLL = 8
J_ITERS = PAIR_VECS // UNROLL
NUM_IDX = SEG_PER_TILE
ACC_PAD = 64
ACC = N * N + ACC_PAD
IMGS_PER_CORE = B // NC
SLICE = N * N // NS
ZCHUNK = 8192


def _adj_body(seg_hbm, out_hbm, segbuf, fwd_a, bwd_a, fwd_b, bwd_b, ones_buf, zbuf, acc, sem, sem2):
    core = lax.axis_index("c")
    sid = lax.axis_index("s")
    iota = lax.iota(jnp.int32, 16)
    ones_v = jnp.full((16,), 1.0, dtype=jnp.float32)
    zero_v = jnp.zeros((16,), dtype=jnp.float32)

    def fill_ones(j, carry):
        ones_buf[pl.ds(j * 16, 16)] = ones_v
        return carry

    lax.fori_loop(0, NUM_IDX // 16, fill_ones, 0)

    def fill_zero(i, carry):
        zbuf[pl.ds(i * 16, 16)] = zero_v
        return carry

    lax.fori_loop(0, ZCHUNK // 16, fill_zero, 0)

    my_slice = pl.multiple_of(sid * SLICE, 8)
    seg_start = pl.multiple_of(sid * SEG_PER_TILE, 8)

    def zero_my_slice():
        descs = [
            pltpu.async_copy(zbuf, acc.at[pl.ds(my_slice + k * ZCHUNK, ZCHUNK)], sem2)
            for k in range(SLICE // ZCHUNK)
        ]

        @pl.when(sid == 0)
        def _zero_pad():
            pltpu.sync_copy(
                zbuf.at[pl.ds(0, ACC_PAD)], acc.at[pl.ds(N * N, ACC_PAD)]
            )

        for d in descs:
            d.wait()

    rows_per_tile_out = N // NS

    def copy_out(img):
        base_row = sid * rows_per_tile_out

        def row_group(g, carry):
            descs = [
                pltpu.async_copy(
                    acc.at[pl.ds(pl.multiple_of((base_row + g * 16 + r) * N, 8), N)],
                    out_hbm.at[img, base_row + g * 16 + r, :],
                    sem2,
                )
                for r in range(16)
            ]
            for d in descs:
                d.wait()
            return carry

        lax.fori_loop(0, rows_per_tile_out // 16, row_group, 0)

    def stage_and_compute(img, fwd_buf, bwd_buf):
        pltpu.sync_copy(
            seg_hbm.at[img, pl.ds(seg_start, SEG_PER_TILE)],
            segbuf.at[pl.ds(0, SEG_PER_TILE)],
        )

        def pair_chunks(j, carry):
            for u in range(UNROLL):
                p = j * (UNROLL * 16) + u * 16
                left = segbuf[pl.ds(pl.multiple_of(p, 16), 16)]
                right = segbuf[pl.ds(p + 1, 16)]
                pos = p + iota
                valid = (left != right) & (lax.rem(pos, W) != (W - 1))
                trash = N * N + iota
                fwd = jnp.where(valid, left * N + right, trash)
                bwd = jnp.where(valid, right * N + left, trash)
                fwd_buf[pl.ds(p, 16)] = fwd
                bwd_buf[pl.ds(p, 16)] = bwd
            return carry

        lax.fori_loop(0, J_ITERS, pair_chunks, 0)

    idx_bufs = [(fwd_a, bwd_a), (fwd_b, bwd_b)]
    stage_and_compute(core * IMGS_PER_CORE, fwd_a, bwd_a)
    zero_my_slice()
    plsc.subcore_barrier()

    for i in range(IMGS_PER_CORE):
        img = core * IMGS_PER_CORE + i

        cur_f, cur_b = idx_bufs[i % 2]
        scat_f = pltpu.async_copy(ones_buf, acc.at[cur_f], sem, add=True)
        scat_b = pltpu.async_copy(ones_buf, acc.at[cur_b], sem, add=True)
        if i + 1 < IMGS_PER_CORE:
            nxt_f, nxt_b = idx_bufs[(i + 1) % 2]
            stage_and_compute(img + 1, nxt_f, nxt_b)
        scat_f.wait()
        scat_b.wait()

        plsc.subcore_barrier()

        copy_out(img)

        if i + 1 < IMGS_PER_CORE:
            zero_my_slice()
            plsc.subcore_barrier()


@jax.jit
def _adjacency(seg_flat):
    mesh = plsc.VectorSubcoreMesh(
        core_axis_name="c", subcore_axis_name="s", num_cores=NC, num_subcores=NS
    )
    run = functools.partial(
        pl.kernel,
        out_type=jax.ShapeDtypeStruct((B, N, N), jnp.float32),
        mesh=mesh,
        scratch_types=[
            pltpu.VMEM((SEG_PER_TILE + 8,), jnp.int32),
            pltpu.VMEM((NUM_IDX,), jnp.int32),
            pltpu.VMEM((NUM_IDX,), jnp.int32),
            pltpu.VMEM((NUM_IDX,), jnp.int32),
            pltpu.VMEM((NUM_IDX,), jnp.int32),
            pltpu.VMEM((NUM_IDX,), jnp.float32),
            pltpu.VMEM((ZCHUNK,), jnp.float32),
            pltpu.VMEM_SHARED((ACC,), jnp.float32),
            pltpu.SemaphoreType.DMA,
            pltpu.SemaphoreType.DMA,
        ],
    )(_adj_body)
    return run(seg_flat)


def kernel(segments):
    if segments.ndim == 4:
        segments = segments[:, 0]
    seg_flat = segments.reshape(B, H * W).astype(jnp.int32)
    return _adjacency(seg_flat)


# speedup vs baseline: 2.8882x; 1.0363x over previous
import functools

import jax
import jax.numpy as jnp
from jax import lax
from jax.experimental import pallas as pl
from jax.experimental.pallas import tpu as pltpu
from jax.experimental.pallas import tpu_sc as plsc

B = 8
H = 384
W = 384
N = 1024
NC = 2
NS = 16
ROWS_PER_TILE = H // NS
SEG_PER_TILE = ROWS_PER_TILE * W
PAIR_VECS = SEG_PER_TILE // 16
UNROLL = 8
J_ITERS = PAIR_VECS // UNROLL
NUM_IDX = SEG_PER_TILE
ACC_PAD = 64
ACC = N * N + ACC_PAD
IMGS_PER_CORE = B // NC
SLICE = N * N // NS
ZCHUNK = 8192


def _adj_body(seg_hbm, out_hbm, segbuf, fwd_a, bwd_a, fwd_b, bwd_b, ones_buf, zbuf, acc, sem, sem2, sem3):
    core = lax.axis_index("c")
    sid = lax.axis_index("s")
    iota = lax.iota(jnp.int32, 16)
    ones_v = jnp.full((16,), 1.0, dtype=jnp.float32)
    zero_v = jnp.zeros((16,), dtype=jnp.float32)

    def fill_ones(j, carry):
        ones_buf[pl.ds(j * 16, 16)] = ones_v
        return carry

    lax.fori_loop(0, NUM_IDX // 16, fill_ones, 0)

    def fill_zero(i, carry):
        zbuf[pl.ds(i * 16, 16)] = zero_v
        return carry

    lax.fori_loop(0, ZCHUNK // 16, fill_zero, 0)

    my_slice = pl.multiple_of(sid * SLICE, 8)
    seg_start = pl.multiple_of(sid * SEG_PER_TILE, 8)

    def zero_my_slice():
        descs = [
            pltpu.async_copy(zbuf, acc.at[pl.ds(my_slice + k * ZCHUNK, ZCHUNK)], sem2)
            for k in range(SLICE // ZCHUNK)
        ]

        @pl.when(sid == 0)
        def _zero_pad():
            pltpu.sync_copy(
                zbuf.at[pl.ds(0, ACC_PAD)], acc.at[pl.ds(N * N, ACC_PAD)]
            )

        for d in descs:
            d.wait()

    rows_per_tile_out = N // NS

    def copy_out_and_zero(img):
        base_row = sid * rows_per_tile_out
        zero_descs = []

        def row_group(g):
            copies = [
                pltpu.async_copy(
                    acc.at[pl.ds(pl.multiple_of((base_row + g * 16 + r) * N, 8), N)],
                    out_hbm.at[img, base_row + g * 16 + r, :],
                    sem2,
                )
                for r in range(16)
            ]
            for d in copies:
                d.wait()
            zero_descs.extend(
                pltpu.async_copy(
                    zbuf,
                    acc.at[pl.ds(my_slice + (2 * g + h) * ZCHUNK, ZCHUNK)],
                    sem3,
                )
                for h in range(2)
            )

        for g in range(rows_per_tile_out // 16):
            row_group(g)

        @pl.when(sid == 0)
        def _zero_pad():
            pltpu.sync_copy(
                zbuf.at[pl.ds(0, ACC_PAD)], acc.at[pl.ds(N * N, ACC_PAD)]
            )

        for d in zero_descs:
            d.wait()

    def copy_out_last(img):
        base_row = sid * rows_per_tile_out

        def row_group(g, carry):
            copies = [
                pltpu.async_copy(
                    acc.at[pl.ds(pl.multiple_of((base_row + g * 16 + r) * N, 8), N)],
                    out_hbm.at[img, base_row + g * 16 + r, :],
                    sem2,
                )
                for r in range(16)
            ]
            for d in copies:
                d.wait()
            return carry

        lax.fori_loop(0, rows_per_tile_out // 16, row_group, 0)

    def stage_and_compute(img, fwd_buf, bwd_buf):
        pltpu.sync_copy(
            seg_hbm.at[img, pl.ds(seg_start, SEG_PER_TILE)],
            segbuf.at[pl.ds(0, SEG_PER_TILE)],
        )

        def pair_chunks(j, carry):
            for u in range(UNROLL):
                p = j * (UNROLL * 16) + u * 16
                left = segbuf[pl.ds(pl.multiple_of(p, 16), 16)]
                right = segbuf[pl.ds(p + 1, 16)]
                pos = p + iota
                valid = (left != right) & (lax.rem(pos, W) != (W - 1))
                trash = N * N + iota
                fwd = jnp.where(valid, left * N + right, trash)
                bwd = jnp.where(valid, right * N + left, trash)
                fwd_buf[pl.ds(p, 16)] = fwd
                bwd_buf[pl.ds(p, 16)] = bwd
            return carry

        lax.fori_loop(0, J_ITERS, pair_chunks, 0)

    idx_bufs = [(fwd_a, bwd_a), (fwd_b, bwd_b)]
    stage_and_compute(core * IMGS_PER_CORE, fwd_a, bwd_a)
    zero_my_slice()
    plsc.subcore_barrier()

    for i in range(IMGS_PER_CORE):
        img = core * IMGS_PER_CORE + i

        cur_f, cur_b = idx_bufs[i % 2]
        scat_f = pltpu.async_copy(ones_buf, acc.at[cur_f], sem, add=True)
        scat_b = pltpu.async_copy(ones_buf, acc.at[cur_b], sem, add=True)
        if i + 1 < IMGS_PER_CORE:
            nxt_f, nxt_b = idx_bufs[(i + 1) % 2]
            stage_and_compute(img + 1, nxt_f, nxt_b)
        scat_f.wait()
        scat_b.wait()

        plsc.subcore_barrier()

        if i + 1 < IMGS_PER_CORE:
            copy_out_and_zero(img)
            plsc.subcore_barrier()
        else:
            copy_out_last(img)


@jax.jit
def _adjacency(seg_flat):
    mesh = plsc.VectorSubcoreMesh(
        core_axis_name="c", subcore_axis_name="s", num_cores=NC, num_subcores=NS
    )
    run = functools.partial(
        pl.kernel,
        out_type=jax.ShapeDtypeStruct((B, N, N), jnp.float32),
        mesh=mesh,
        scratch_types=[
            pltpu.VMEM((SEG_PER_TILE + 8,), jnp.int32),
            pltpu.VMEM((NUM_IDX,), jnp.int32),
            pltpu.VMEM((NUM_IDX,), jnp.int32),
            pltpu.VMEM((NUM_IDX,), jnp.int32),
            pltpu.VMEM((NUM_IDX,), jnp.int32),
            pltpu.VMEM((NUM_IDX,), jnp.float32),
            pltpu.VMEM((ZCHUNK,), jnp.float32),
            pltpu.VMEM_SHARED((ACC,), jnp.float32),
            pltpu.SemaphoreType.DMA,
            pltpu.SemaphoreType.DMA,
            pltpu.SemaphoreType.DMA,
        ],
    )(_adj_body)
    return run(seg_flat)


def kernel(segments):
    if segments.ndim == 4:
        segments = segments[:, 0]
    seg_flat = segments.reshape(B, H * W).astype(jnp.int32)
    return _adjacency(seg_flat)


# speedup vs baseline: 2.9747x; 1.0300x over previous
import functools

import jax
import jax.numpy as jnp
from jax import lax
from jax.experimental import pallas as pl
from jax.experimental.pallas import tpu as pltpu
from jax.experimental.pallas import tpu_sc as plsc

B = 8
H = 384
W = 384
N = 1024
NC = 2
NS = 16
ROWS_PER_TILE = H // NS
SEG_PER_TILE = ROWS_PER_TILE * W
PAIR_VECS = SEG_PER_TILE // 16
UNROLL = 8
J_ITERS = PAIR_VECS // UNROLL
NUM_IDX = SEG_PER_TILE
ACC_PAD = 64
ACC = N * N + ACC_PAD
IMGS_PER_CORE = B // NC
SLICE = N * N // NS
ZCHUNK = 8192


def _adj_body(seg_hbm, out_hbm, segbuf, fwd_a, bwd_a, fwd_b, bwd_b, ones_buf, zbuf, acc, sem, sem2, sem3):
    core = lax.axis_index("c")
    sid = lax.axis_index("s")
    iota = lax.iota(jnp.int32, 16)
    ones_v = jnp.full((16,), 1.0, dtype=jnp.float32)
    zero_v = jnp.zeros((16,), dtype=jnp.float32)

    def fill_ones(j, carry):
        ones_buf[pl.ds(j * 16, 16)] = ones_v
        return carry

    lax.fori_loop(0, NUM_IDX // 16, fill_ones, 0)

    def fill_zero(i, carry):
        zbuf[pl.ds(i * 16, 16)] = zero_v
        return carry

    lax.fori_loop(0, ZCHUNK // 16, fill_zero, 0)

    my_slice = pl.multiple_of(sid * SLICE, 8)
    seg_start = pl.multiple_of(sid * SEG_PER_TILE, 8)

    def start_zero_my_slice():
        return [
            pltpu.async_copy(zbuf, acc.at[pl.ds(my_slice + k * ZCHUNK, ZCHUNK)], sem3)
            for k in range(SLICE // ZCHUNK)
        ]

    rows_per_tile_out = N // NS

    def copy_out_and_zero(img):
        base_row = sid * rows_per_tile_out
        zero_descs = []

        def row_group(g):
            copies = [
                pltpu.async_copy(
                    acc.at[pl.ds(pl.multiple_of((base_row + g * 16 + r) * N, 8), N)],
                    out_hbm.at[img, base_row + g * 16 + r, :],
                    sem2,
                )
                for r in range(16)
            ]
            for d in copies:
                d.wait()
            zero_descs.extend(
                pltpu.async_copy(
                    zbuf,
                    acc.at[pl.ds(my_slice + (2 * g + h) * ZCHUNK, ZCHUNK)],
                    sem3,
                )
                for h in range(2)
            )

        for g in range(rows_per_tile_out // 16):
            row_group(g)

        for d in zero_descs:
            d.wait()

    def copy_out_last(img):
        base_row = sid * rows_per_tile_out

        def row_group(g, carry):
            copies = [
                pltpu.async_copy(
                    acc.at[pl.ds(pl.multiple_of((base_row + g * 16 + r) * N, 8), N)],
                    out_hbm.at[img, base_row + g * 16 + r, :],
                    sem2,
                )
                for r in range(16)
            ]
            for d in copies:
                d.wait()
            return carry

        lax.fori_loop(0, rows_per_tile_out // 16, row_group, 0)

    def stage_and_compute(img, fwd_buf, bwd_buf):
        pltpu.sync_copy(
            seg_hbm.at[img, pl.ds(seg_start, SEG_PER_TILE)],
            segbuf.at[pl.ds(0, SEG_PER_TILE)],
        )

        def pair_chunks(j, carry):
            for u in range(UNROLL):
                p = j * (UNROLL * 16) + u * 16
                left = segbuf[pl.ds(pl.multiple_of(p, 16), 16)]
                right = segbuf[pl.ds(p + 1, 16)]
                pos = p + iota
                valid = (left != right) & (lax.rem(pos, W) != (W - 1))
                trash = N * N + iota
                fwd = jnp.where(valid, left * N + right, trash)
                bwd = jnp.where(valid, right * N + left, trash)
                fwd_buf[pl.ds(p, 16)] = fwd
                bwd_buf[pl.ds(p, 16)] = bwd
            return carry

        lax.fori_loop(0, J_ITERS, pair_chunks, 0)

    idx_bufs = [(fwd_a, bwd_a), (fwd_b, bwd_b)]
    zero_descs = start_zero_my_slice()
    stage_and_compute(core * IMGS_PER_CORE, fwd_a, bwd_a)
    for d in zero_descs:
        d.wait()
    plsc.subcore_barrier()

    for i in range(IMGS_PER_CORE):
        img = core * IMGS_PER_CORE + i

        cur_f, cur_b = idx_bufs[i % 2]
        scat_f = pltpu.async_copy(ones_buf, acc.at[cur_f], sem, add=True)
        scat_b = pltpu.async_copy(ones_buf, acc.at[cur_b], sem, add=True)
        if i + 1 < IMGS_PER_CORE:
            nxt_f, nxt_b = idx_bufs[(i + 1) % 2]
            stage_and_compute(img + 1, nxt_f, nxt_b)
        scat_f.wait()
        scat_b.wait()

        plsc.subcore_barrier()

        if i + 1 < IMGS_PER_CORE:
            copy_out_and_zero(img)
            plsc.subcore_barrier()
        else:
            copy_out_last(img)


@jax.jit
def _adjacency(seg_flat):
    mesh = plsc.VectorSubcoreMesh(
        core_axis_name="c", subcore_axis_name="s", num_cores=NC, num_subcores=NS
    )
    run = functools.partial(
        pl.kernel,
        out_type=jax.ShapeDtypeStruct((B, N, N), jnp.float32),
        mesh=mesh,
        scratch_types=[
            pltpu.VMEM((SEG_PER_TILE + 8,), jnp.int32),
            pltpu.VMEM((NUM_IDX,), jnp.int32),
            pltpu.VMEM((NUM_IDX,), jnp.int32),
            pltpu.VMEM((NUM_IDX,), jnp.int32),
            pltpu.VMEM((NUM_IDX,), jnp.int32),
            pltpu.VMEM((NUM_IDX,), jnp.float32),
            pltpu.VMEM((ZCHUNK,), jnp.float32),
            pltpu.VMEM_SHARED((ACC,), jnp.float32),
            pltpu.SemaphoreType.DMA,
            pltpu.SemaphoreType.DMA,
            pltpu.SemaphoreType.DMA,
        ],
    )(_adj_body)
    return run(seg_flat)


def kernel(segments):
    if segments.ndim == 4:
        segments = segments[:, 0]
    seg_flat = segments.reshape(B, H * W).astype(jnp.int32)
    return _adjacency(seg_flat)


# speedup vs baseline: 3.0656x; 1.0306x over previous
import functools

import jax
import jax.numpy as jnp
from jax import lax
from jax.experimental import pallas as pl
from jax.experimental.pallas import tpu as pltpu
from jax.experimental.pallas import tpu_sc as plsc

B = 8
H = 384
W = 384
N = 1024
NC = 2
NS = 16
ROWS_PER_TILE = H // NS
SEG_PER_TILE = ROWS_PER_TILE * W
PAIR_VECS = SEG_PER_TILE // 16
UNROLL = 8
J_ITERS = PAIR_VECS // UNROLL
NUM_IDX = SEG_PER_TILE
ACC_PAD = 64
ACC = N * N + ACC_PAD
IMGS_PER_CORE = B // NC
SLICE = N * N // NS
ZCHUNK = 8192


def _adj_body(seg_hbm, out_hbm, segbuf, fwd_a, bwd_a, fwd_b, bwd_b, ones_buf, zbuf, acc, sem, sem2, sem3):
    core = lax.axis_index("c")
    sid = lax.axis_index("s")
    iota = lax.iota(jnp.int32, 16)
    ones_v = jnp.full((16,), 1.0, dtype=jnp.float32)
    zero_v = jnp.zeros((16,), dtype=jnp.float32)

    def fill_ones(j, carry):
        ones_buf[pl.ds(j * 16, 16)] = ones_v
        return carry

    lax.fori_loop(0, NUM_IDX // 16, fill_ones, 0)

    def fill_zero(i, carry):
        zbuf[pl.ds(i * 16, 16)] = zero_v
        return carry

    lax.fori_loop(0, ZCHUNK // 16, fill_zero, 0)

    my_slice = pl.multiple_of(sid * SLICE, 8)
    seg_start = pl.multiple_of(sid * SEG_PER_TILE, 8)

    def start_zero_my_slice():
        return [
            pltpu.async_copy(zbuf, acc.at[pl.ds(my_slice + k * ZCHUNK, ZCHUNK)], sem3)
            for k in range(SLICE // ZCHUNK)
        ]

    rows_per_tile_out = N // NS

    def copy_out_and_zero(img):
        base_row = sid * rows_per_tile_out
        zero_descs = []

        def row_group(g):
            copies = [
                pltpu.async_copy(
                    acc.at[pl.ds(pl.multiple_of((base_row + g * 16 + r) * N, 8), N)],
                    out_hbm.at[img, base_row + g * 16 + r, :],
                    sem2,
                )
                for r in range(16)
            ]
            for d in copies:
                d.wait()
            zero_descs.extend(
                pltpu.async_copy(
                    zbuf,
                    acc.at[pl.ds(my_slice + (2 * g + h) * ZCHUNK, ZCHUNK)],
                    sem3,
                )
                for h in range(2)
            )

        for g in range(rows_per_tile_out // 16):
            row_group(g)

        for d in zero_descs:
            d.wait()

    def copy_out_last(img):
        base_row = sid * rows_per_tile_out

        def row_group(g, carry):
            copies = [
                pltpu.async_copy(
                    acc.at[pl.ds(pl.multiple_of((base_row + g * 16 + r) * N, 8), N)],
                    out_hbm.at[img, base_row + g * 16 + r, :],
                    sem2,
                )
                for r in range(16)
            ]
            for d in copies:
                d.wait()
            return carry

        lax.fori_loop(0, rows_per_tile_out // 16, row_group, 0)

    def stage_and_compute(img, fwd_buf, bwd_buf):
        pltpu.sync_copy(
            seg_hbm.at[img, pl.ds(seg_start, SEG_PER_TILE)],
            segbuf.at[pl.ds(0, SEG_PER_TILE)],
        )

        @plsc.parallel_loop(0, PAIR_VECS, 1, unroll=UNROLL)
        def pair_chunks(j):
            p = j * 16
            left = segbuf[pl.ds(pl.multiple_of(p, 16), 16)]
            right = segbuf[pl.ds(p + 1, 16)]
            pos = p + iota
            valid = (left != right) & (lax.rem(pos, W) != (W - 1))
            trash = N * N + iota
            fwd = jnp.where(valid, left * N + right, trash)
            bwd = jnp.where(valid, right * N + left, trash)
            fwd_buf[pl.ds(p, 16)] = fwd
            bwd_buf[pl.ds(p, 16)] = bwd

    idx_bufs = [(fwd_a, bwd_a), (fwd_b, bwd_b)]
    zero_descs = start_zero_my_slice()
    stage_and_compute(core * IMGS_PER_CORE, fwd_a, bwd_a)
    for d in zero_descs:
        d.wait()
    plsc.subcore_barrier()

    for i in range(IMGS_PER_CORE):
        img = core * IMGS_PER_CORE + i

        cur_f, cur_b = idx_bufs[i % 2]
        scat_f = pltpu.async_copy(ones_buf, acc.at[cur_f], sem, add=True)
        scat_b = pltpu.async_copy(ones_buf, acc.at[cur_b], sem, add=True)
        if i + 1 < IMGS_PER_CORE:
            nxt_f, nxt_b = idx_bufs[(i + 1) % 2]
            stage_and_compute(img + 1, nxt_f, nxt_b)
        scat_f.wait()
        scat_b.wait()

        plsc.subcore_barrier()

        if i + 1 < IMGS_PER_CORE:
            copy_out_and_zero(img)
            plsc.subcore_barrier()
        else:
            copy_out_last(img)


@jax.jit
def _adjacency(seg_flat):
    mesh = plsc.VectorSubcoreMesh(
        core_axis_name="c", subcore_axis_name="s", num_cores=NC, num_subcores=NS
    )
    run = functools.partial(
        pl.kernel,
        out_type=jax.ShapeDtypeStruct((B, N, N), jnp.float32),
        mesh=mesh,
        scratch_types=[
            pltpu.VMEM((SEG_PER_TILE + 8,), jnp.int32),
            pltpu.VMEM((NUM_IDX,), jnp.int32),
            pltpu.VMEM((NUM_IDX,), jnp.int32),
            pltpu.VMEM((NUM_IDX,), jnp.int32),
            pltpu.VMEM((NUM_IDX,), jnp.int32),
            pltpu.VMEM((NUM_IDX,), jnp.float32),
            pltpu.VMEM((ZCHUNK,), jnp.float32),
            pltpu.VMEM_SHARED((ACC,), jnp.float32),
            pltpu.SemaphoreType.DMA,
            pltpu.SemaphoreType.DMA,
            pltpu.SemaphoreType.DMA,
        ],
    )(_adj_body)
    return run(seg_flat)


def kernel(segments):
    if segments.ndim == 4:
        segments = segments[:, 0]
    seg_flat = segments.reshape(B, H * W).astype(jnp.int32)
    return _adjacency(seg_flat)


# speedup vs baseline: 3.1510x; 1.0279x over previous
import functools

import jax
import jax.numpy as jnp
from jax import lax
from jax.experimental import pallas as pl
from jax.experimental.pallas import tpu as pltpu
from jax.experimental.pallas import tpu_sc as plsc

B = 8
H = 384
W = 384
N = 1024
NC = 2
NS = 16
ROWS_PER_TILE = H // NS
SEG_PER_TILE = ROWS_PER_TILE * W
PAIR_VECS = SEG_PER_TILE // 16
UNROLL = 8
J_ITERS = PAIR_VECS // UNROLL
NUM_IDX = SEG_PER_TILE
ACC_PAD = 64
ACC = N * N + ACC_PAD
IMGS_PER_CORE = B // NC
SLICE = N * N // NS
ZCHUNK = 8192


def _adj_body(seg_hbm, out_hbm, segbuf, fwd_a, bwd_a, fwd_b, bwd_b, ones_buf, zbuf, acc, sem, sem2, sem3):
    core = lax.axis_index("c")
    sid = lax.axis_index("s")
    iota = lax.iota(jnp.int32, 16)
    ones_v = jnp.full((16,), 1.0, dtype=jnp.float32)
    zero_v = jnp.zeros((16,), dtype=jnp.float32)

    def fill_ones(j, carry):
        ones_buf[pl.ds(j * 16, 16)] = ones_v
        return carry

    lax.fori_loop(0, NUM_IDX // 16, fill_ones, 0)

    def fill_zero(i, carry):
        zbuf[pl.ds(i * 16, 16)] = zero_v
        return carry

    lax.fori_loop(0, ZCHUNK // 16, fill_zero, 0)

    my_slice = pl.multiple_of(sid * SLICE, 8)
    seg_start = pl.multiple_of(sid * SEG_PER_TILE, 8)

    def start_zero_my_slice():
        return [
            pltpu.async_copy(zbuf, acc.at[pl.ds(my_slice + k * ZCHUNK, ZCHUNK)], sem3)
            for k in range(SLICE // ZCHUNK)
        ]

    rows_per_tile_out = N // NS

    def copy_out_and_zero(img):
        base_row = sid * rows_per_tile_out
        n_groups = rows_per_tile_out // 16

        def fire_group(g):
            return [
                pltpu.async_copy(
                    acc.at[pl.ds(pl.multiple_of((base_row + g * 16 + r) * N, 8), N)],
                    out_hbm.at[img, base_row + g * 16 + r, :],
                    sem2,
                )
                for r in range(16)
            ]

        groups = {0: fire_group(0), 1: fire_group(1)}
        zero_descs = []
        for g in range(n_groups):
            for d in groups.pop(g):
                d.wait()
            if g + 2 < n_groups:
                groups[g + 2] = fire_group(g + 2)
            zero_descs.extend(
                pltpu.async_copy(
                    zbuf,
                    acc.at[pl.ds(my_slice + (2 * g + h) * ZCHUNK, ZCHUNK)],
                    sem3,
                )
                for h in range(2)
            )

        for d in zero_descs:
            d.wait()

    def copy_out_last(img):
        base_row = sid * rows_per_tile_out

        def row_group(g, carry):
            copies = [
                pltpu.async_copy(
                    acc.at[pl.ds(pl.multiple_of((base_row + g * 16 + r) * N, 8), N)],
                    out_hbm.at[img, base_row + g * 16 + r, :],
                    sem2,
                )
                for r in range(16)
            ]
            for d in copies:
                d.wait()
            return carry

        lax.fori_loop(0, rows_per_tile_out // 16, row_group, 0)

    def stage_and_compute(img, fwd_buf, bwd_buf):
        pltpu.sync_copy(
            seg_hbm.at[img, pl.ds(seg_start, SEG_PER_TILE)],
            segbuf.at[pl.ds(0, SEG_PER_TILE)],
        )

        @plsc.parallel_loop(0, PAIR_VECS, 1, unroll=UNROLL)
        def pair_chunks(j):
            p = j * 16
            left = segbuf[pl.ds(pl.multiple_of(p, 16), 16)]
            right = segbuf[pl.ds(p + 1, 16)]
            pos = p + iota
            valid = (left != right) & (lax.rem(pos, W) != (W - 1))
            trash = N * N + iota
            fwd = jnp.where(valid, left * N + right, trash)
            bwd = jnp.where(valid, right * N + left, trash)
            fwd_buf[pl.ds(p, 16)] = fwd
            bwd_buf[pl.ds(p, 16)] = bwd

    idx_bufs = [(fwd_a, bwd_a), (fwd_b, bwd_b)]
    zero_descs = start_zero_my_slice()
    stage_and_compute(core * IMGS_PER_CORE, fwd_a, bwd_a)
    for d in zero_descs:
        d.wait()
    plsc.subcore_barrier()

    for i in range(IMGS_PER_CORE):
        img = core * IMGS_PER_CORE + i

        cur_f, cur_b = idx_bufs[i % 2]
        scat_f = pltpu.async_copy(ones_buf, acc.at[cur_f], sem, add=True)
        scat_b = pltpu.async_copy(ones_buf, acc.at[cur_b], sem, add=True)
        if i + 1 < IMGS_PER_CORE:
            nxt_f, nxt_b = idx_bufs[(i + 1) % 2]
            stage_and_compute(img + 1, nxt_f, nxt_b)
        scat_f.wait()
        scat_b.wait()

        plsc.subcore_barrier()

        if i + 1 < IMGS_PER_CORE:
            copy_out_and_zero(img)
            plsc.subcore_barrier()
        else:
            copy_out_last(img)


@jax.jit
def _adjacency(seg_flat):
    mesh = plsc.VectorSubcoreMesh(
        core_axis_name="c", subcore_axis_name="s", num_cores=NC, num_subcores=NS
    )
    run = functools.partial(
        pl.kernel,
        out_type=jax.ShapeDtypeStruct((B, N, N), jnp.float32),
        mesh=mesh,
        scratch_types=[
            pltpu.VMEM((SEG_PER_TILE + 8,), jnp.int32),
            pltpu.VMEM((NUM_IDX,), jnp.int32),
            pltpu.VMEM((NUM_IDX,), jnp.int32),
            pltpu.VMEM((NUM_IDX,), jnp.int32),
            pltpu.VMEM((NUM_IDX,), jnp.int32),
            pltpu.VMEM((NUM_IDX,), jnp.float32),
            pltpu.VMEM((ZCHUNK,), jnp.float32),
            pltpu.VMEM_SHARED((ACC,), jnp.float32),
            pltpu.SemaphoreType.DMA,
            pltpu.SemaphoreType.DMA,
            pltpu.SemaphoreType.DMA,
        ],
    )(_adj_body)
    return run(seg_flat)


def kernel(segments):
    if segments.ndim == 4:
        segments = segments[:, 0]
    seg_flat = segments.reshape(B, H * W).astype(jnp.int32)
    return _adjacency(seg_flat)


# speedup vs baseline: 3.1643x; 1.0042x over previous
import functools

import jax
import jax.numpy as jnp
from jax import lax
from jax.experimental import pallas as pl
from jax.experimental.pallas import tpu as pltpu
from jax.experimental.pallas import tpu_sc as plsc

B = 8
H = 384
W = 384
N = 1024
NC = 2
NS = 16
ROWS_PER_TILE = H // NS
SEG_PER_TILE = ROWS_PER_TILE * W
PAIR_VECS = SEG_PER_TILE // 16
UNROLL = 16
J_ITERS = PAIR_VECS // UNROLL
NUM_IDX = SEG_PER_TILE
ACC_PAD = 64
ACC = N * N + ACC_PAD
IMGS_PER_CORE = B // NC
SLICE = N * N // NS
ZCHUNK = 8192


def _adj_body(seg_hbm, out_hbm, segbuf, fwd_a, bwd_a, fwd_b, bwd_b, ones_buf, zbuf, acc, sem, sem2, sem3):
    core = lax.axis_index("c")
    sid = lax.axis_index("s")
    iota = lax.iota(jnp.int32, 16)
    ones_v = jnp.full((16,), 1.0, dtype=jnp.float32)
    zero_v = jnp.zeros((16,), dtype=jnp.float32)

    def fill_ones(j, carry):
        ones_buf[pl.ds(j * 16, 16)] = ones_v
        return carry

    lax.fori_loop(0, NUM_IDX // 16, fill_ones, 0)

    def fill_zero(i, carry):
        zbuf[pl.ds(i * 16, 16)] = zero_v
        return carry

    lax.fori_loop(0, ZCHUNK // 16, fill_zero, 0)

    my_slice = pl.multiple_of(sid * SLICE, 8)
    seg_start = pl.multiple_of(sid * SEG_PER_TILE, 8)

    def start_zero_my_slice():
        return [
            pltpu.async_copy(zbuf, acc.at[pl.ds(my_slice + k * ZCHUNK, ZCHUNK)], sem3)
            for k in range(SLICE // ZCHUNK)
        ]

    rows_per_tile_out = N // NS

    def copy_out_and_zero(img):
        base_row = sid * rows_per_tile_out
        n_groups = rows_per_tile_out // 16

        def fire_group(g):
            return [
                pltpu.async_copy(
                    acc.at[pl.ds(pl.multiple_of((base_row + g * 16 + r) * N, 8), N)],
                    out_hbm.at[img, base_row + g * 16 + r, :],
                    sem2,
                )
                for r in range(16)
            ]

        groups = {0: fire_group(0), 1: fire_group(1)}
        zero_descs = []
        for g in range(n_groups):
            for d in groups.pop(g):
                d.wait()
            if g + 2 < n_groups:
                groups[g + 2] = fire_group(g + 2)
            zero_descs.extend(
                pltpu.async_copy(
                    zbuf,
                    acc.at[pl.ds(my_slice + (2 * g + h) * ZCHUNK, ZCHUNK)],
                    sem3,
                )
                for h in range(2)
            )

        for d in zero_descs:
            d.wait()

    def copy_out_last(img):
        base_row = sid * rows_per_tile_out
        copies = [
            pltpu.async_copy(
                acc.at[pl.ds(pl.multiple_of((base_row + r) * N, 8), N)],
                out_hbm.at[img, base_row + r, :],
                sem2,
            )
            for r in range(rows_per_tile_out)
        ]
        for d in copies:
            d.wait()

    def stage_and_compute(img, fwd_buf, bwd_buf):
        pltpu.sync_copy(
            seg_hbm.at[img, pl.ds(seg_start, SEG_PER_TILE)],
            segbuf.at[pl.ds(0, SEG_PER_TILE)],
        )

        @plsc.parallel_loop(0, PAIR_VECS, 1, unroll=UNROLL)
        def pair_chunks(j):
            p = j * 16
            left = segbuf[pl.ds(pl.multiple_of(p, 16), 16)]
            right = segbuf[pl.ds(p + 1, 16)]
            pos = p + iota
            valid = (left != right) & (lax.rem(pos, W) != (W - 1))
            trash = N * N + iota
            fwd = jnp.where(valid, left * N + right, trash)
            bwd = jnp.where(valid, right * N + left, trash)
            fwd_buf[pl.ds(p, 16)] = fwd
            bwd_buf[pl.ds(p, 16)] = bwd

    idx_bufs = [(fwd_a, bwd_a), (fwd_b, bwd_b)]
    zero_descs = start_zero_my_slice()
    stage_and_compute(core * IMGS_PER_CORE, fwd_a, bwd_a)
    for d in zero_descs:
        d.wait()
    plsc.subcore_barrier()

    for i in range(IMGS_PER_CORE):
        img = core * IMGS_PER_CORE + i

        cur_f, cur_b = idx_bufs[i % 2]
        scat_f = pltpu.async_copy(ones_buf, acc.at[cur_f], sem, add=True)
        scat_b = pltpu.async_copy(ones_buf, acc.at[cur_b], sem, add=True)
        if i + 1 < IMGS_PER_CORE:
            nxt_f, nxt_b = idx_bufs[(i + 1) % 2]
            stage_and_compute(img + 1, nxt_f, nxt_b)
        scat_f.wait()
        scat_b.wait()

        plsc.subcore_barrier()

        if i + 1 < IMGS_PER_CORE:
            copy_out_and_zero(img)
            plsc.subcore_barrier()
        else:
            copy_out_last(img)


@jax.jit
def _adjacency(seg_flat):
    mesh = plsc.VectorSubcoreMesh(
        core_axis_name="c", subcore_axis_name="s", num_cores=NC, num_subcores=NS
    )
    run = functools.partial(
        pl.kernel,
        out_type=jax.ShapeDtypeStruct((B, N, N), jnp.float32),
        mesh=mesh,
        scratch_types=[
            pltpu.VMEM((SEG_PER_TILE + 8,), jnp.int32),
            pltpu.VMEM((NUM_IDX,), jnp.int32),
            pltpu.VMEM((NUM_IDX,), jnp.int32),
            pltpu.VMEM((NUM_IDX,), jnp.int32),
            pltpu.VMEM((NUM_IDX,), jnp.int32),
            pltpu.VMEM((NUM_IDX,), jnp.float32),
            pltpu.VMEM((ZCHUNK,), jnp.float32),
            pltpu.VMEM_SHARED((ACC,), jnp.float32),
            pltpu.SemaphoreType.DMA,
            pltpu.SemaphoreType.DMA,
            pltpu.SemaphoreType.DMA,
        ],
    )(_adj_body)
    return run(seg_flat)


def kernel(segments):
    if segments.ndim == 4:
        segments = segments[:, 0]
    seg_flat = segments.reshape(B, H * W).astype(jnp.int32)
    return _adjacency(seg_flat)
